# initial kernel scaffold (unmeasured)
import jax
import jax.numpy as jnp
from jax import lax
from jax.experimental import pallas as pl
from jax.experimental.pallas import tpu as pltpu


def kernel(
    x,
):
    def body(*refs):
        pass

    out_shape = jax.ShapeDtypeStruct(..., jnp.float32)
    return pl.pallas_call(body, out_shape=out_shape)(...)


# baseline (device time: 49682 ns/iter reference)
import jax
import jax.numpy as jnp
from jax import lax
from jax.experimental import pallas as pl
from jax.experimental.pallas import tpu as pltpu

N_DEV = 4


def kernel(x):
    m, n = x.shape
    ch = m // N_DEV

    def body(x_ref, out_ref, sbuf, comm, send_sems, recv_sems):
        my = lax.axis_index("i")
        left = (my + N_DEV - 1) % N_DEV
        right = (my + 1) % N_DEV

        barrier_sem = pltpu.get_barrier_semaphore()
        for nbr in (left, right):
            pl.semaphore_signal(
                barrier_sem, inc=1,
                device_id=(nbr,), device_id_type=pl.DeviceIdType.MESH,
            )
        pl.semaphore_wait(barrier_sem, 2)

        sbuf[0, :, :] = x_ref[pl.ds(my * ch, ch), :]
        for h in range(N_DEV - 1):
            rdma = pltpu.make_async_remote_copy(
                src_ref=sbuf.at[h],
                dst_ref=comm.at[h],
                send_sem=send_sems.at[h],
                recv_sem=recv_sems.at[h],
                device_id=(right,),
                device_id_type=pl.DeviceIdType.MESH,
            )
            rdma.start()
            rdma.wait()
            c_recv = (my + N_DEV - 1 - h) % N_DEV
            acc = comm[h, :, :] + x_ref[pl.ds(c_recv * ch, ch), :]
            sbuf[h + 1, :, :] = acc
            if h == N_DEV - 2:
                out_ref[pl.ds(c_recv * ch, ch), :] = acc

        for g in range(N_DEV - 1):
            src = sbuf.at[N_DEV - 1] if g == 0 else comm.at[N_DEV - 1 + g - 1]
            s = N_DEV - 1 + g
            rdma = pltpu.make_async_remote_copy(
                src_ref=src,
                dst_ref=comm.at[s],
                send_sem=send_sems.at[s],
                recv_sem=recv_sems.at[s],
                device_id=(right,),
                device_id_type=pl.DeviceIdType.MESH,
            )
            rdma.start()
            rdma.wait()
            origin = (my + N_DEV - g) % N_DEV
            out_ref[pl.ds(origin * ch, ch), :] = comm[s, :, :]

    return pl.pallas_call(
        body,
        out_shape=jax.ShapeDtypeStruct((m, n), x.dtype),
        in_specs=[pl.BlockSpec(memory_space=pltpu.VMEM)],
        out_specs=pl.BlockSpec(memory_space=pltpu.VMEM),
        scratch_shapes=[
            pltpu.VMEM((N_DEV, ch, n), x.dtype),
            pltpu.VMEM((2 * N_DEV - 2, ch, n), x.dtype),
            pltpu.SemaphoreType.DMA((2 * N_DEV - 2,)),
            pltpu.SemaphoreType.DMA((2 * N_DEV - 2,)),
        ],
        compiler_params=pltpu.CompilerParams(collective_id=0),
    )(x)


# device time: 33038 ns/iter; 1.5038x vs baseline; 1.5038x over previous
import jax
import jax.numpy as jnp
from jax import lax
from jax.experimental import pallas as pl
from jax.experimental.pallas import tpu as pltpu

N_DEV = 4
N_HOPS = 2 * (N_DEV - 1)


def kernel(x):
    m, n = x.shape
    ch = m // N_DEV
    half = n // 2

    def body(x_ref, out_ref, sbuf, comm, send_sems, recv_sems):
        my = lax.axis_index("i")
        left = (my + N_DEV - 1) % N_DEV
        right = (my + 1) % N_DEV
        dst = [right, left]
        cols = [slice(0, half), slice(half, n)]

        barrier_sem = pltpu.get_barrier_semaphore()
        for nbr in (left, right):
            pl.semaphore_signal(
                barrier_sem, inc=1,
                device_id=(nbr,), device_id_type=pl.DeviceIdType.MESH,
            )
        pl.semaphore_wait(barrier_sem, 2)

        rdmas = []

        def hop(d, slot, src_ref):
            r = pltpu.make_async_remote_copy(
                src_ref=src_ref,
                dst_ref=comm.at[d, slot],
                send_sem=send_sems.at[d, slot],
                recv_sem=recv_sems.at[d, slot],
                device_id=(dst[d],),
                device_id_type=pl.DeviceIdType.MESH,
            )
            r.start()
            rdmas.append(r)
            return r

        def rs_recv_chunk(d, h):
            off = -(h + 1) if d == 0 else (h + 1)
            return (my + off + N_DEV) % N_DEV

        def ag_origin_chunk(d, g):
            off = -g if d == 0 else g
            return (my + off + N_DEV) % N_DEV

        sbuf[0, 0, :, :] = x_ref[pl.ds(my * ch, ch), cols[0]]
        sbuf[1, 0, :, :] = x_ref[pl.ds(my * ch, ch), cols[1]]
        for h in range(N_DEV - 1):
            rs = [hop(d, h, sbuf.at[d, h]) for d in range(2)]
            for d in range(2):
                rs[d].wait_recv()
            for d in range(2):
                c = rs_recv_chunk(d, h)
                acc = comm[d, h, :, :] + x_ref[pl.ds(c * ch, ch), cols[d]]
                sbuf[d, h + 1, :, :] = acc
                if h == N_DEV - 2:
                    out_ref[pl.ds(c * ch, ch), cols[d]] = acc

        for g in range(N_DEV - 1):
            s = N_DEV - 1 + g
            rs = [
                hop(d, s, sbuf.at[d, N_DEV - 1] if g == 0 else comm.at[d, s - 1])
                for d in range(2)
            ]
            for d in range(2):
                rs[d].wait_recv()
            for d in range(2):
                c = ag_origin_chunk(d, g)
                out_ref[pl.ds(c * ch, ch), cols[d]] = comm[d, s, :, :]

        for r in rdmas:
            r.wait_send()

    return pl.pallas_call(
        body,
        out_shape=jax.ShapeDtypeStruct((m, n), x.dtype),
        in_specs=[pl.BlockSpec(memory_space=pltpu.VMEM)],
        out_specs=pl.BlockSpec(memory_space=pltpu.VMEM),
        scratch_shapes=[
            pltpu.VMEM((2, N_DEV, ch, half), x.dtype),
            pltpu.VMEM((2, N_HOPS, ch, half), x.dtype),
            pltpu.SemaphoreType.DMA((2, N_HOPS)),
            pltpu.SemaphoreType.DMA((2, N_HOPS)),
        ],
        compiler_params=pltpu.CompilerParams(collective_id=0),
    )(x)


# device time: 25512 ns/iter; 1.9474x vs baseline; 1.2950x over previous
import jax
import jax.numpy as jnp
from jax import lax
from jax.experimental import pallas as pl
from jax.experimental.pallas import tpu as pltpu

N_DEV = 4
N_HOPS = 2 * (N_DEV - 1)
N_SUB = 4


def kernel(x):
    m, n = x.shape
    ch = m // N_DEV
    half = n // 2
    sub = ch // N_SUB

    def body(x_ref, out_ref, sbuf, comm, send_sems, recv_sems):
        my = lax.axis_index("i")
        left = (my + N_DEV - 1) % N_DEV
        right = (my + 1) % N_DEV
        dst = [right, left]
        cols = [slice(0, half), slice(half, n)]

        barrier_sem = pltpu.get_barrier_semaphore()
        for nbr in (left, right):
            pl.semaphore_signal(
                barrier_sem, inc=1,
                device_id=(nbr,), device_id_type=pl.DeviceIdType.MESH,
            )
        pl.semaphore_wait(barrier_sem, 2)

        rdmas = {}

        def start_send(d, h, s, src_ref):
            r = pltpu.make_async_remote_copy(
                src_ref=src_ref,
                dst_ref=comm.at[d, h, pl.ds(s * sub, sub)],
                send_sem=send_sems.at[d, h, s],
                recv_sem=recv_sems.at[d, h, s],
                device_id=(dst[d],),
                device_id_type=pl.DeviceIdType.MESH,
            )
            r.start()
            rdmas[(d, h, s)] = r
            return r

        def rs_recv_chunk(d, h):
            off = -(h + 1) if d == 0 else (h + 1)
            return (my + off + N_DEV) % N_DEV

        def ag_origin_chunk(d, g):
            off = -g if d == 0 else g
            return (my + off + N_DEV) % N_DEV

        for d in range(2):
            sbuf[d, 0, :, :] = x_ref[pl.ds(my * ch, ch), cols[d]]
        for s in range(N_SUB):
            for d in range(2):
                start_send(d, 0, s, sbuf.at[d, 0, pl.ds(s * sub, sub)])

        for h in range(N_HOPS):
            for s in range(N_SUB):
                for d in range(2):
                    rdmas[(d, h, s)].wait_recv()
                for d in range(2):
                    rows = pl.ds(s * sub, sub)
                    if h < N_DEV - 1:
                        c = rs_recv_chunk(d, h)
                        grows = pl.ds(c * ch + s * sub, sub)
                        acc = comm[d, h, rows, :] + x_ref[grows, cols[d]]
                        if h < N_DEV - 2:
                            sbuf[d, h + 1, rows, :] = acc
                            start_send(d, h + 1, s,
                                       sbuf.at[d, h + 1, rows])
                        else:
                            out_ref[grows, cols[d]] = acc
                            sbuf[d, h + 1, rows, :] = acc
                            start_send(d, h + 1, s,
                                       sbuf.at[d, h + 1, rows])
                    else:
                        g = h - (N_DEV - 1)
                        c = ag_origin_chunk(d, g)
                        grows = pl.ds(c * ch + s * sub, sub)
                        out_ref[grows, cols[d]] = comm[d, h, rows, :]
                        if h < N_HOPS - 1:
                            start_send(d, h + 1, s, comm.at[d, h, rows])

        for r in rdmas.values():
            r.wait_send()

    return pl.pallas_call(
        body,
        out_shape=jax.ShapeDtypeStruct((m, n), x.dtype),
        in_specs=[pl.BlockSpec(memory_space=pltpu.VMEM)],
        out_specs=pl.BlockSpec(memory_space=pltpu.VMEM),
        scratch_shapes=[
            pltpu.VMEM((2, N_DEV, ch, half), x.dtype),
            pltpu.VMEM((2, N_HOPS, ch, half), x.dtype),
            pltpu.SemaphoreType.DMA((2, N_HOPS, N_SUB)),
            pltpu.SemaphoreType.DMA((2, N_HOPS, N_SUB)),
        ],
        compiler_params=pltpu.CompilerParams(collective_id=0),
    )(x)


# device time: 25471 ns/iter; 1.9505x vs baseline; 1.0016x over previous
import jax
import jax.numpy as jnp
from jax import lax
from jax.experimental import pallas as pl
from jax.experimental.pallas import tpu as pltpu

N_DEV = 4
N_HOPS = 2 * (N_DEV - 1)
N_SUB = 4


def kernel(x):
    m, n = x.shape
    ch = m // N_DEV
    half = n // 2
    sub = ch // N_SUB

    def body(x_ref, out_ref, sbuf, comm, send_sems, recv_sems):
        my = lax.axis_index("i")
        left = (my + N_DEV - 1) % N_DEV
        right = (my + 1) % N_DEV
        dst = [right, left]
        cols = [slice(0, half), slice(half, n)]

        barrier_sem = pltpu.get_barrier_semaphore()
        for nbr in (left, right):
            pl.semaphore_signal(
                barrier_sem, inc=1,
                device_id=(nbr,), device_id_type=pl.DeviceIdType.MESH,
            )
        pl.semaphore_wait(barrier_sem, 2)

        rdmas = {}

        def start_send(d, h, s, src_ref):
            r = pltpu.make_async_remote_copy(
                src_ref=src_ref,
                dst_ref=comm.at[d, h, pl.ds(s * sub, sub)],
                send_sem=send_sems.at[d, h, s],
                recv_sem=recv_sems.at[d, h, s],
                device_id=(dst[d],),
                device_id_type=pl.DeviceIdType.MESH,
            )
            r.start()
            rdmas[(d, h, s)] = r
            return r

        def rs_recv_chunk(d, h):
            off = -(h + 1) if d == 0 else (h + 1)
            return (my + off + N_DEV) % N_DEV

        def ag_origin_chunk(d, g):
            off = -g if d == 0 else g
            return (my + off + N_DEV) % N_DEV

        for s in range(N_SUB):
            for d in range(2):
                start_send(d, 0, s,
                           x_ref.at[pl.ds(my * ch + s * sub, sub), cols[d]])

        for h in range(N_HOPS):
            for s in range(N_SUB):
                for d in range(2):
                    rdmas[(d, h, s)].wait_recv()
                for d in range(2):
                    rows = pl.ds(s * sub, sub)
                    if h < N_DEV - 1:
                        c = rs_recv_chunk(d, h)
                        grows = pl.ds(c * ch + s * sub, sub)
                        acc = comm[d, h, rows, :] + x_ref[grows, cols[d]]
                        if h < N_DEV - 2:
                            sbuf[d, h + 1, rows, :] = acc
                            start_send(d, h + 1, s,
                                       sbuf.at[d, h + 1, rows])
                        else:
                            out_ref[grows, cols[d]] = acc
                            start_send(d, h + 1, s,
                                       out_ref.at[grows, cols[d]])
                    else:
                        g = h - (N_DEV - 1)
                        c = ag_origin_chunk(d, g)
                        grows = pl.ds(c * ch + s * sub, sub)
                        if h < N_HOPS - 1:
                            start_send(d, h + 1, s, comm.at[d, h, rows])
                        out_ref[grows, cols[d]] = comm[d, h, rows, :]

        for r in rdmas.values():
            r.wait_send()

    return pl.pallas_call(
        body,
        out_shape=jax.ShapeDtypeStruct((m, n), x.dtype),
        in_specs=[pl.BlockSpec(memory_space=pltpu.VMEM)],
        out_specs=pl.BlockSpec(memory_space=pltpu.VMEM),
        scratch_shapes=[
            pltpu.VMEM((2, N_DEV, ch, half), x.dtype),
            pltpu.VMEM((2, N_HOPS, ch, half), x.dtype),
            pltpu.SemaphoreType.DMA((2, N_HOPS, N_SUB)),
            pltpu.SemaphoreType.DMA((2, N_HOPS, N_SUB)),
        ],
        compiler_params=pltpu.CompilerParams(collective_id=0),
    )(x)


# device time: 25296 ns/iter; 1.9640x vs baseline; 1.0069x over previous
import jax
import jax.numpy as jnp
from jax import lax
from jax.experimental import pallas as pl
from jax.experimental.pallas import tpu as pltpu

N_DEV = 4
N_SUB = 4


def kernel(x):
    m, n = x.shape
    ch = m // N_DEV
    half = n // 2
    sub = ch // N_SUB

    def body(x_ref, out_ref, abuf, mbuf, rbuf,
             a_ss, a_rs, b_ss, b_rs, g_ss, g_rs):
        my = lax.axis_index("i")
        left = (my + N_DEV - 1) % N_DEV
        right = (my + 1) % N_DEV
        colL = slice(0, half)
        colH = slice(half, n)
        col = [colL, colH]

        def crows(rel, s):
            return pl.ds(((my + rel + N_DEV) % N_DEV) * ch + s * sub, sub)

        def srows(s):
            return pl.ds(s * sub, sub)

        barrier_sem = pltpu.get_barrier_semaphore()
        for b in (left, right):
            pl.semaphore_signal(
                barrier_sem, inc=1,
                device_id=(b,), device_id_type=pl.DeviceIdType.MESH,
            )
        pl.semaphore_wait(barrier_sem, 2)

        all_rdmas = []

        def send(src_ref, dst_ref, ssem, rsem, dev):
            r = pltpu.make_async_remote_copy(
                src_ref=src_ref, dst_ref=dst_ref,
                send_sem=ssem, recv_sem=rsem,
                device_id=(dev,), device_id_type=pl.DeviceIdType.MESH,
            )
            r.start()
            all_rdmas.append(r)

        def wait_recv(shape_ref, ssem, rsem):
            r = pltpu.make_async_remote_copy(
                src_ref=shape_ref, dst_ref=shape_ref,
                send_sem=ssem, recv_sem=rsem,
                device_id=(right,), device_id_type=pl.DeviceIdType.MESH,
            )
            r.wait_recv()

        for s in range(N_SUB):
            send(x_ref.at[crows(2, s), colL], abuf.at[0, srows(s)],
                 a_ss.at[0, s], a_rs.at[0, s], right)
            send(x_ref.at[crows(2, s), colH], abuf.at[1, srows(s)],
                 a_ss.at[1, s], a_rs.at[1, s], left)
            send(x_ref.at[crows(1, s), colH], rbuf.at[0, 1, srows(s)],
                 b_ss.at[0, 1, s], b_rs.at[0, 1, s], right)
            send(x_ref.at[crows(-1, s), colL], rbuf.at[1, 0, srows(s)],
                 b_ss.at[1, 0, s], b_rs.at[1, 0, s], left)

        for s in range(N_SUB):
            wait_recv(abuf.at[0, srows(s)], a_ss.at[0, s], a_rs.at[0, s])
            mbuf[0, srows(s), :] = (
                abuf[0, srows(s), :] + x_ref[crows(1, s), colL]
            )
            send(mbuf.at[0, srows(s)], rbuf.at[0, 0, srows(s)],
                 b_ss.at[0, 0, s], b_rs.at[0, 0, s], right)
            wait_recv(abuf.at[1, srows(s)], a_ss.at[1, s], a_rs.at[1, s])
            mbuf[1, srows(s), :] = (
                abuf[1, srows(s), :] + x_ref[crows(-1, s), colH]
            )
            send(mbuf.at[1, srows(s)], rbuf.at[1, 1, srows(s)],
                 b_ss.at[1, 1, s], b_rs.at[1, 1, s], left)

        for s in range(N_SUB):
            for side in range(2):
                for part in range(2):
                    wait_recv(rbuf.at[side, part, srows(s)],
                              b_ss.at[side, part, s],
                              b_rs.at[side, part, s])
            for part in range(2):
                out_ref[crows(0, s), col[part]] = (
                    x_ref[crows(0, s), col[part]]
                    + rbuf[0, part, srows(s), :]
                    + rbuf[1, part, srows(s), :]
                )
            send(out_ref.at[crows(0, s), :], out_ref.at[crows(0, s), :],
                 g_ss.at[0, s], g_rs.at[0, s], right)
            send(out_ref.at[crows(0, s), :], out_ref.at[crows(0, s), :],
                 g_ss.at[1, s], g_rs.at[1, s], left)

        for s in range(N_SUB):
            wait_recv(out_ref.at[srows(s), :], g_ss.at[0, s], g_rs.at[0, s])
            send(out_ref.at[crows(-1, s), colL],
                 out_ref.at[crows(-1, s), colL],
                 g_ss.at[2, s], g_rs.at[2, s], right)
            wait_recv(out_ref.at[srows(s), :], g_ss.at[1, s], g_rs.at[1, s])
            send(out_ref.at[crows(1, s), colH],
                 out_ref.at[crows(1, s), colH],
                 g_ss.at[3, s], g_rs.at[3, s], left)

        for s in range(N_SUB):
            wait_recv(out_ref.at[srows(s), colL], g_ss.at[2, s], g_rs.at[2, s])
            wait_recv(out_ref.at[srows(s), colH], g_ss.at[3, s], g_rs.at[3, s])

        for r in all_rdmas:
            r.wait_send()

    return pl.pallas_call(
        body,
        out_shape=jax.ShapeDtypeStruct((m, n), x.dtype),
        in_specs=[pl.BlockSpec(memory_space=pltpu.VMEM)],
        out_specs=pl.BlockSpec(memory_space=pltpu.VMEM),
        scratch_shapes=[
            pltpu.VMEM((2, ch, half), x.dtype),
            pltpu.VMEM((2, ch, half), x.dtype),
            pltpu.VMEM((2, 2, ch, half), x.dtype),
            pltpu.SemaphoreType.DMA((2, N_SUB)),
            pltpu.SemaphoreType.DMA((2, N_SUB)),
            pltpu.SemaphoreType.DMA((2, 2, N_SUB)),
            pltpu.SemaphoreType.DMA((2, 2, N_SUB)),
            pltpu.SemaphoreType.DMA((4, N_SUB)),
            pltpu.SemaphoreType.DMA((4, N_SUB)),
        ],
        compiler_params=pltpu.CompilerParams(collective_id=0),
    )(x)
